# baseline (speedup 1.0000x reference)
import functools

import jax
import jax.numpy as jnp
from jax import lax
from jax.experimental import pallas as pl
from jax.experimental.pallas import tpu as pltpu
from jax.experimental.pallas import tpu_sc as plsc

N = 2048
D = 1024
E = 16
H = 2048
C = 256
DUMP = E * C
EI_ROWS = DUMP + 8
EO_ROWS = (E + 1) * C
THRESH = 0.2
EPS = 1e-9

EH = E // 2
HALF = EH * C
EO_HALF = (EH + 1) * C

NC = 2
NS = 16
NW = NC * NS
TPW = N // NW


def _cumsum0(a):
    c = a
    k = 1
    while k < N:
        c = c + jnp.concatenate([jnp.zeros((k, E), c.dtype), c[:-k]], axis=0)
        k *= 2
    return c


def _gating_body(x_ref, wg_ref, probs_ref, s1_ref, s2_ref, c1_ref, c2_ref):
    x = x_ref[...]
    wg = wg_ref[...]
    logits = jnp.dot(x, wg, preferred_element_type=jnp.float32)
    m = jnp.max(logits, axis=-1, keepdims=True)
    un = jnp.exp(logits - m)
    raw = un / jnp.sum(un, axis=-1, keepdims=True)

    iota_e = lax.broadcasted_iota(jnp.int32, (N, E), 1)
    g1 = jnp.max(raw, axis=-1)
    i1 = jnp.min(jnp.where(raw == g1[:, None], iota_e, E), axis=-1)
    m1 = (iota_e == i1[:, None]).astype(jnp.float32)
    wo1 = raw * (1.0 - m1)
    g2 = jnp.max(wo1, axis=-1)
    i2 = jnp.min(jnp.where(wo1 == g2[:, None], iota_e, E), axis=-1)
    m2 = (iota_e == i2[:, None]).astype(jnp.float32)

    denom = g1 + g2 + EPS
    g1n = g1 / denom
    g2n = g2 / denom
    keep2 = (probs_ref[...][:, 0] < (g2n / THRESH)).astype(jnp.float32)
    m2 = m2 * keep2[:, None]

    cum1 = _cumsum0(m1)
    pos1m = (cum1 - m1) * m1
    m1c = m1 * (pos1m < float(C)).astype(jnp.float32)
    m1_count = jnp.sum(m1c, axis=0)
    m1_flat = jnp.sum(m1c, axis=1)
    pos1_flat = jnp.sum(pos1m, axis=1)
    g1f = g1n * m1_flat

    cum2 = _cumsum0(m2)
    pos2m = (cum2 - m2 + m1_count[None, :]) * m2
    m2c = m2 * (pos2m < float(C)).astype(jnp.float32)
    m2_flat = jnp.sum(m2c, axis=1)
    pos2_flat = jnp.sum(pos2m, axis=1)
    g2f = g2n * m2_flat

    slot1 = jnp.where(m1_flat > 0, i1 * C + pos1_flat.astype(jnp.int32), DUMP)
    slot2 = jnp.where(m2_flat > 0, i2 * C + pos2_flat.astype(jnp.int32), DUMP)

    s1_ref[...] = slot1[:, None]
    s2_ref[...] = slot2[:, None]
    c1_ref[...] = jnp.broadcast_to(g1f[:, None], (N, E))
    c2_ref[...] = jnp.broadcast_to(g2f[:, None], (N, E))


_gating = pl.pallas_call(
    _gating_body,
    out_shape=[
        jax.ShapeDtypeStruct((N, 1), jnp.int32),
        jax.ShapeDtypeStruct((N, 1), jnp.int32),
        jax.ShapeDtypeStruct((N, E), jnp.float32),
        jax.ShapeDtypeStruct((N, E), jnp.float32),
    ],
)


CHUNK = 16
NCHUNK = TPW // CHUNK


@functools.cache
def _sc_kernels():
    mesh = plsc.VectorSubcoreMesh(core_axis_name="c", subcore_axis_name="s")

    @functools.partial(
        pl.kernel,
        mesh=mesh,
        out_type=jax.ShapeDtypeStruct((EI_ROWS, D), jnp.float32),
        scratch_types=[
            pltpu.VMEM((TPW,), jnp.int32),
            pltpu.VMEM((TPW,), jnp.int32),
            pltpu.VMEM((TPW, D), jnp.float32),
            pltpu.SemaphoreType.DMA,
            pltpu.SemaphoreType.DMA,
            pltpu.SemaphoreType.DMA,
        ],
    )
    def _dispatch(x_hbm, s1_hbm, s2_hbm, ei_hbm, i1v, i2v, xbuf,
                  sema, semb, semc):
        wid = lax.axis_index("s") * NC + lax.axis_index("c")
        base = wid * TPW
        cpa = pltpu.async_copy(s1_hbm.at[pl.ds(base, TPW)], i1v, sema)
        cpb = pltpu.async_copy(s2_hbm.at[pl.ds(base, TPW)], i2v, semb)
        cpc = pltpu.async_copy(x_hbm.at[pl.ds(base, TPW)], xbuf, semc)
        cpa.wait()
        cpb.wait()
        cpc.wait()
        cp1 = pltpu.async_copy(xbuf, ei_hbm.at[i1v], sema)
        cp2 = pltpu.async_copy(xbuf, ei_hbm.at[i2v], semb)
        cp1.wait()
        cp2.wait()

    combine_scratch = [
        pltpu.VMEM((TPW, E), jnp.float32),
        pltpu.VMEM((TPW, E), jnp.float32),
        pltpu.VMEM((TPW,), jnp.int32),
        pltpu.VMEM((TPW,), jnp.int32),
        pltpu.VMEM((CHUNK, D), jnp.float32),
        pltpu.VMEM((CHUNK, D), jnp.float32),
        pltpu.VMEM((CHUNK, D), jnp.float32),
        pltpu.VMEM((CHUNK, D), jnp.float32),
        pltpu.VMEM((CHUNK, D), jnp.float32),
        pltpu.VMEM((CHUNK, D), jnp.float32),
    ] + [pltpu.SemaphoreType.DMA] * 8

    def _combine_body(part, eo_hbm, s1_hbm, s2_hbm, c1_hbm, c2_hbm, p_hbm,
                      out_hbm, w1v, w2v, i1v, i2v, r1a, r1b, r2a, r2b,
                      aca, acb, s1a, s1b, s2a, s2b, spa, spb, swa, swb):
        wid = lax.axis_index("s") * NC + lax.axis_index("c")
        base = wid * TPW
        r1 = [r1a, r1b]
        r2 = [r2a, r2b]
        ac = [aca, acb]
        sg1 = [s1a, s1b]
        sg2 = [s2a, s2b]
        sp = [spa, spb]
        sw = [swa, swb]
        cpa = pltpu.async_copy(c1_hbm.at[pl.ds(base, TPW)], w1v, swa)
        cpb = pltpu.async_copy(c2_hbm.at[pl.ds(base, TPW)], w2v, swb)
        cpc = pltpu.async_copy(s1_hbm.at[pl.ds(base, TPW)], i1v, sg1[0])
        cpd = pltpu.async_copy(s2_hbm.at[pl.ds(base, TPW)], i2v, sg2[0])
        cpa.wait()
        cpb.wait()
        cpc.wait()
        cpd.wait()
        for iv in (i1v, i2v):
            for k in range(TPW // 16):
                v = iv[pl.ds(k * 16, 16)]
                if part == 0:
                    t = jnp.where(v < HALF, v, HALF)
                else:
                    t = jnp.where(v >= HALF, v - HALF, HALF)
                iv[pl.ds(k * 16, 16)] = t

        def gathers(ch, b):
            lo = ch * CHUNK
            cps = [
                pltpu.async_copy(
                    eo_hbm.at[i1v.at[pl.ds(lo, CHUNK)]], r1[b], sg1[b]),
                pltpu.async_copy(
                    eo_hbm.at[i2v.at[pl.ds(lo, CHUNK)]], r2[b], sg2[b]),
            ]
            if part == 1:
                cps.append(pltpu.async_copy(
                    p_hbm.at[pl.ds(base + lo, CHUNK)], ac[b], sp[b]))
            return cps

        g = gathers(0, 0)
        wcp = [None, None]
        for ch in range(NCHUNK):
            b = ch & 1
            cur = g
            if ch + 1 < NCHUNK:
                if wcp[1 - b] is not None:
                    wcp[1 - b].wait()
                    wcp[1 - b] = None
                g = gathers(ch + 1, 1 - b)
            for cp in cur:
                cp.wait()
            if wcp[b] is not None:
                wcp[b].wait()
                wcp[b] = None
            for j in range(CHUNK):
                a = w1v[ch * CHUNK + j, :]
                c = w2v[ch * CHUNK + j, :]

                def vbody(v, _, j=j, a=a, c=c, b=b):
                    off = v * 128
                    for u in range(8):
                        o = off + u * 16
                        s = a * r1[b][j, pl.ds(o, 16)] + c * r2[b][j, pl.ds(o, 16)]
                        if part == 1:
                            s = ac[b][j, pl.ds(o, 16)] + s
                        ac[b][j, pl.ds(o, 16)] = s
                    return 0

                lax.fori_loop(0, D // 128, vbody, 0)
            wcp[b] = pltpu.async_copy(
                ac[b], out_hbm.at[pl.ds(base + ch * CHUNK, CHUNK)], sw[b])
        for b in (0, 1):
            if wcp[b] is not None:
                wcp[b].wait()

    @functools.partial(
        pl.kernel,
        mesh=mesh,
        out_type=jax.ShapeDtypeStruct((N, D), jnp.float32),
        scratch_types=combine_scratch,
    )
    def _combine_a(eo_hbm, s1_hbm, s2_hbm, c1_hbm, c2_hbm, out_hbm, *scr):
        _combine_body(0, eo_hbm, s1_hbm, s2_hbm, c1_hbm, c2_hbm, None,
                      out_hbm, *scr)

    @functools.partial(
        pl.kernel,
        mesh=mesh,
        out_type=jax.ShapeDtypeStruct((N, D), jnp.float32),
        scratch_types=combine_scratch,
    )
    def _combine_b(eo_hbm, s1_hbm, s2_hbm, c1_hbm, c2_hbm, p_hbm, out_hbm,
                   *scr):
        _combine_body(1, eo_hbm, s1_hbm, s2_hbm, c1_hbm, c2_hbm, p_hbm,
                      out_hbm, *scr)

    return _dispatch, _combine_a, _combine_b


HT = 2048
NH = H // HT


def _ffn_body(x_ref, w1_ref, w2_ref, o_ref):
    e = pl.program_id(0)
    h = pl.program_id(1)

    @pl.when(h == 0)
    def _init():
        o_ref[...] = jnp.zeros_like(o_ref)

    @pl.when(e < EH)
    def _compute():
        xb = x_ref[...].astype(jnp.bfloat16)
        w1b = w1_ref[0].astype(jnp.bfloat16)
        hid = jnp.maximum(
            jnp.dot(xb, w1b, preferred_element_type=jnp.float32), 0.0
        ).astype(jnp.bfloat16)
        w2b = w2_ref[0].astype(jnp.bfloat16)
        o_ref[...] += jnp.dot(hid, w2b, preferred_element_type=jnp.float32)


def _make_ffn(part):
    def emap(e, h, part=part):
        return (jnp.minimum(part * EH + e, E - 1), 0)

    return pl.pallas_call(
        _ffn_body,
        grid=(EH + 1, NH),
        in_specs=[
            pl.BlockSpec((C, D), emap),
            pl.BlockSpec((1, D, HT),
                         lambda e, h, part=part:
                         (jnp.minimum(part * EH + e, E - 1), 0, h)),
            pl.BlockSpec((1, HT, D),
                         lambda e, h, part=part:
                         (jnp.minimum(part * EH + e, E - 1), h, 0)),
        ],
        out_specs=pl.BlockSpec((C, D), lambda e, h: (e, 0)),
        out_shape=jax.ShapeDtypeStruct((EO_HALF, D), jnp.float32),
        compiler_params=pltpu.CompilerParams(
            dimension_semantics=("arbitrary", "arbitrary"),
        ),
    )


_ffn_a = _make_ffn(0)
_ffn_b = _make_ffn(1)


def kernel(inputs, w_gating, w1, w2):
    x = inputs.reshape(N, D)
    probs = jax.random.uniform(jax.random.key(42), (1, N), dtype=jnp.float32)
    probs2d = probs.reshape(N, 1)
    s1r, s2r, c1r, c2r = _gating(x, w_gating, probs2d)
    slot1 = s1r.reshape(N)
    slot2 = s2r.reshape(N)
    dispatch_k, combine_a, combine_b = _sc_kernels()
    ei = dispatch_k(x, slot1, slot2)
    eo_a = _ffn_a(ei, w1, w2)
    eo_b = _ffn_b(ei, w1, w2)
    partial = combine_a(eo_a, slot1, slot2, c1r, c2r)
    out = combine_b(eo_b, slot1, slot2, c1r, c2r, partial)
    return out.reshape(1, N, D)


# speedup vs baseline: 2.2673x; 2.2673x over previous
import functools

import jax
import jax.numpy as jnp
from jax import lax
from jax.experimental import pallas as pl
from jax.experimental.pallas import tpu as pltpu
from jax.experimental.pallas import tpu_sc as plsc

N = 2048
D = 1024
E = 16
H = 2048
C = 256
DUMP = E * C
EI_ROWS = DUMP + 8
EO_ROWS = (E + 1) * C
THRESH = 0.2
EPS = 1e-9

NC = 2
NS = 16
NW = NC * NS
TPW = N // NW


def _cumsum0(a):
    c = a
    k = 1
    while k < N:
        c = c + jnp.concatenate([jnp.zeros((k, E), c.dtype), c[:-k]], axis=0)
        k *= 2
    return c


def _gating_body(x_ref, wg_ref, probs_ref, s1_ref, s2_ref, c1_ref, c2_ref):
    x = x_ref[...]
    wg = wg_ref[...]
    logits = jnp.dot(x, wg, preferred_element_type=jnp.float32)
    m = jnp.max(logits, axis=-1, keepdims=True)
    un = jnp.exp(logits - m)
    raw = un / jnp.sum(un, axis=-1, keepdims=True)

    iota_e = lax.broadcasted_iota(jnp.int32, (N, E), 1)
    g1 = jnp.max(raw, axis=-1)
    i1 = jnp.min(jnp.where(raw == g1[:, None], iota_e, E), axis=-1)
    m1 = (iota_e == i1[:, None]).astype(jnp.float32)
    wo1 = raw * (1.0 - m1)
    g2 = jnp.max(wo1, axis=-1)
    i2 = jnp.min(jnp.where(wo1 == g2[:, None], iota_e, E), axis=-1)
    m2 = (iota_e == i2[:, None]).astype(jnp.float32)

    denom = g1 + g2 + EPS
    g1n = g1 / denom
    g2n = g2 / denom
    keep2 = (probs_ref[...][:, 0] < (g2n / THRESH)).astype(jnp.float32)
    m2 = m2 * keep2[:, None]

    cum1 = _cumsum0(m1)
    pos1m = (cum1 - m1) * m1
    m1c = m1 * (pos1m < float(C)).astype(jnp.float32)
    m1_count = jnp.sum(m1c, axis=0)
    m1_flat = jnp.sum(m1c, axis=1)
    pos1_flat = jnp.sum(pos1m, axis=1)
    g1f = g1n * m1_flat

    cum2 = _cumsum0(m2)
    pos2m = (cum2 - m2 + m1_count[None, :]) * m2
    m2c = m2 * (pos2m < float(C)).astype(jnp.float32)
    m2_flat = jnp.sum(m2c, axis=1)
    pos2_flat = jnp.sum(pos2m, axis=1)
    g2f = g2n * m2_flat

    slot1 = jnp.where(m1_flat > 0, i1 * C + pos1_flat.astype(jnp.int32), DUMP)
    slot2 = jnp.where(m2_flat > 0, i2 * C + pos2_flat.astype(jnp.int32), DUMP)

    s1_ref[...] = slot1[:, None]
    s2_ref[...] = slot2[:, None]
    c1_ref[...] = jnp.broadcast_to(g1f[:, None], (N, E))
    c2_ref[...] = jnp.broadcast_to(g2f[:, None], (N, E))


_gating = pl.pallas_call(
    _gating_body,
    out_shape=[
        jax.ShapeDtypeStruct((N, 1), jnp.int32),
        jax.ShapeDtypeStruct((N, 1), jnp.int32),
        jax.ShapeDtypeStruct((N, E), jnp.float32),
        jax.ShapeDtypeStruct((N, E), jnp.float32),
    ],
)


CHUNK = 16
NCHUNK = TPW // CHUNK


@functools.cache
def _sc_kernels():
    mesh = plsc.VectorSubcoreMesh(core_axis_name="c", subcore_axis_name="s")

    @functools.partial(
        pl.kernel,
        mesh=mesh,
        out_type=jax.ShapeDtypeStruct((EI_ROWS, D), jnp.float32),
        scratch_types=[
            pltpu.VMEM((TPW,), jnp.int32),
            pltpu.VMEM((TPW,), jnp.int32),
            pltpu.VMEM((TPW, D), jnp.float32),
            pltpu.SemaphoreType.DMA,
            pltpu.SemaphoreType.DMA,
            pltpu.SemaphoreType.DMA,
        ],
    )
    def _dispatch(x_hbm, s1_hbm, s2_hbm, ei_hbm, i1v, i2v, xbuf,
                  sema, semb, semc):
        wid = lax.axis_index("s") * NC + lax.axis_index("c")
        base = wid * TPW
        cpa = pltpu.async_copy(s1_hbm.at[pl.ds(base, TPW)], i1v, sema)
        cpb = pltpu.async_copy(s2_hbm.at[pl.ds(base, TPW)], i2v, semb)
        cpc = pltpu.async_copy(x_hbm.at[pl.ds(base, TPW)], xbuf, semc)
        cpa.wait()
        cpb.wait()
        cpc.wait()
        cp1 = pltpu.async_copy(xbuf, ei_hbm.at[i1v], sema)
        cp2 = pltpu.async_copy(xbuf, ei_hbm.at[i2v], semb)
        cp1.wait()
        cp2.wait()

    @functools.partial(
        pl.kernel,
        mesh=mesh,
        out_type=jax.ShapeDtypeStruct((N, D), jnp.float32),
        scratch_types=[
            pltpu.VMEM((TPW, E), jnp.float32),
            pltpu.VMEM((TPW, E), jnp.float32),
            pltpu.VMEM((TPW,), jnp.int32),
            pltpu.VMEM((TPW,), jnp.int32),
            pltpu.VMEM((CHUNK, D), jnp.float32),
            pltpu.VMEM((CHUNK, D), jnp.float32),
            pltpu.VMEM((CHUNK, D), jnp.float32),
            pltpu.VMEM((CHUNK, D), jnp.float32),
            pltpu.VMEM((CHUNK, D), jnp.float32),
            pltpu.VMEM((CHUNK, D), jnp.float32),
            pltpu.SemaphoreType.DMA,
            pltpu.SemaphoreType.DMA,
            pltpu.SemaphoreType.DMA,
            pltpu.SemaphoreType.DMA,
            pltpu.SemaphoreType.DMA,
            pltpu.SemaphoreType.DMA,
        ],
    )
    def _combine(eo_hbm, s1_hbm, s2_hbm, c1_hbm, c2_hbm, out_hbm,
                 w1v, w2v, i1v, i2v, r1a, r1b, r2a, r2b, oba, obb,
                 s1a, s1b, s2a, s2b, swa, swb):
        wid = lax.axis_index("s") * NC + lax.axis_index("c")
        base = wid * TPW
        r1 = [r1a, r1b]
        r2 = [r2a, r2b]
        ob = [oba, obb]
        sg1 = [s1a, s1b]
        sg2 = [s2a, s2b]
        sw = [swa, swb]
        cpa = pltpu.async_copy(c1_hbm.at[pl.ds(base, TPW)], w1v, swa)
        cpb = pltpu.async_copy(c2_hbm.at[pl.ds(base, TPW)], w2v, swb)
        cpc = pltpu.async_copy(s1_hbm.at[pl.ds(base, TPW)], i1v, sg1[0])
        cpd = pltpu.async_copy(s2_hbm.at[pl.ds(base, TPW)], i2v, sg2[0])
        cpa.wait()
        cpb.wait()
        cpc.wait()
        cpd.wait()

        def gathers(ch, b):
            lo = ch * CHUNK
            cp1 = pltpu.async_copy(
                eo_hbm.at[i1v.at[pl.ds(lo, CHUNK)]], r1[b], sg1[b])
            cp2 = pltpu.async_copy(
                eo_hbm.at[i2v.at[pl.ds(lo, CHUNK)]], r2[b], sg2[b])
            return cp1, cp2

        g = gathers(0, 0)
        wcp = [None, None]
        for ch in range(NCHUNK):
            b = ch & 1
            cp1, cp2 = g
            if ch + 1 < NCHUNK:
                g = gathers(ch + 1, 1 - b)
            cp1.wait()
            cp2.wait()
            if wcp[b] is not None:
                wcp[b].wait()
            for j in range(CHUNK):
                a = w1v[ch * CHUNK + j, :]
                c = w2v[ch * CHUNK + j, :]

                def vbody(v, _, j=j, a=a, c=c, b=b):
                    off = v * 128
                    for u in range(8):
                        o = off + u * 16
                        ob[b][j, pl.ds(o, 16)] = (a * r1[b][j, pl.ds(o, 16)]
                                                  + c * r2[b][j, pl.ds(o, 16)])
                    return 0

                lax.fori_loop(0, D // 128, vbody, 0)
            wcp[b] = pltpu.async_copy(
                ob[b], out_hbm.at[pl.ds(base + ch * CHUNK, CHUNK)], sw[b])
        wcp[0].wait()
        wcp[1].wait()

    return _dispatch, _combine


HT = 2048
NH = H // HT


def _ffn_body(x_ref, w1_ref, w2_ref, o_ref):
    e = pl.program_id(0)
    h = pl.program_id(1)

    @pl.when(h == 0)
    def _init():
        o_ref[...] = jnp.zeros_like(o_ref)

    @pl.when(e < E)
    def _compute():
        xb = x_ref[...].astype(jnp.bfloat16)
        w1b = w1_ref[0].astype(jnp.bfloat16)
        hid = jnp.maximum(
            jnp.dot(xb, w1b, preferred_element_type=jnp.float32), 0.0
        ).astype(jnp.bfloat16)
        w2b = w2_ref[0].astype(jnp.bfloat16)
        o_ref[...] += jnp.dot(hid, w2b, preferred_element_type=jnp.float32)


_ffn = pl.pallas_call(
    _ffn_body,
    grid=(E + 1, NH),
    in_specs=[
        pl.BlockSpec((C, D), lambda e, h: (jnp.minimum(e, E - 1), 0)),
        pl.BlockSpec((1, D, HT), lambda e, h: (jnp.minimum(e, E - 1), 0, h)),
        pl.BlockSpec((1, HT, D), lambda e, h: (jnp.minimum(e, E - 1), h, 0)),
    ],
    out_specs=pl.BlockSpec((C, D), lambda e, h: (e, 0)),
    out_shape=jax.ShapeDtypeStruct((EO_ROWS, D), jnp.float32),
    compiler_params=pltpu.CompilerParams(
        dimension_semantics=("arbitrary", "arbitrary"),
    ),
)


def kernel(inputs, w_gating, w1, w2):
    x = inputs.reshape(N, D)
    probs = jax.random.uniform(jax.random.key(42), (1, N), dtype=jnp.float32)
    probs2d = probs.reshape(N, 1)
    s1r, s2r, c1r, c2r = _gating(x, w_gating, probs2d)
    slot1 = s1r.reshape(N)
    slot2 = s2r.reshape(N)
    dispatch_k, combine_k = _sc_kernels()
    ei = dispatch_k(x, slot1, slot2)
    eo = _ffn(ei, w1, w2)
    out = combine_k(eo, slot1, slot2, c1r, c2r)
    return out.reshape(1, N, D)
